# baseline (speedup 1.0000x reference)
import jax
import jax.numpy as jnp
from jax import lax
from jax.experimental import pallas as pl
from jax.experimental.pallas import tpu as pltpu

_OVERLAP_THRESH = 0.5
_NEG_POS_RATIO = 3
_VAR0, _VAR1 = 0.1, 0.2

_SUB = 8


def _row_kernel(truths_ref, truths_t_ref, priors_ref, loc_ref, conf_ref,
                out_l_ref, out_c_ref,
                mneg_ref, part_ref):
    b = pl.program_id(0)
    B = pl.num_programs(0)
    tT = truths_t_ref[0]
    pR = priors_ref[...]
    NOBJ = truths_ref.shape[1]
    SUB, LN = pR.shape[1], pR.shape[2]
    P = SUB * LN

    pcx, pcy, pw, ph = pR[0], pR[1], pR[2], pR[3]
    px0 = pcx - pw / 2.0
    py0 = pcy - ph / 2.0
    px1 = pcx + pw / 2.0
    py1 = pcy + ph / 2.0
    area_p = (px1 - px0) * (py1 - py0)

    p_idx = (lax.broadcasted_iota(jnp.int32, (SUB, LN), 0) * LN
             + lax.broadcasted_iota(jnp.int32, (SUB, LN), 1))

    run_bto = None
    run_bti = None
    bpis = []
    for o in range(NOBJ):
        tx0 = truths_ref[0, o, 0]
        ty0 = truths_ref[0, o, 1]
        tx1 = truths_ref[0, o, 2]
        ty1 = truths_ref[0, o, 3]
        at = (tx1 - tx0) * (ty1 - ty0)
        iw = jnp.maximum(jnp.minimum(tx1, px1) - jnp.maximum(tx0, px0), 0.0)
        ih = jnp.maximum(jnp.minimum(ty1, py1) - jnp.maximum(ty0, py0), 0.0)
        inter = iw * ih
        ov_o = inter / ((at + area_p) - inter)
        if o == 0:
            run_bto = ov_o
            run_bti = jnp.zeros((SUB, LN), jnp.int32)
        else:
            upd = ov_o > run_bto
            run_bti = jnp.where(upd, o, run_bti)
            run_bto = jnp.maximum(ov_o, run_bto)
        mo = jnp.max(ov_o)
        bpis.append(jnp.min(jnp.where(ov_o == mo, p_idx, P)))

    scat_o = jnp.full((SUB, LN), -1, jnp.int32)
    for o in range(NOBJ):
        scat_o = jnp.where(p_idx == bpis[o], o, scat_o)
    scat_any = scat_o >= 0
    bti = jnp.where(scat_any, scat_o, run_bti)

    pos = scat_any | jnp.logical_not(run_bto < _OVERLAP_THRESH)
    posf = pos.astype(jnp.float32)

    o_iota = lax.broadcasted_iota(jnp.int32, (NOBJ, SUB, LN), 0)

    onehot = (o_iota == bti[None]).astype(jnp.float32)
    matched = lax.dot_general(tT, onehot,
                              dimension_numbers=(((1,), (0,)), ((), ())),
                              preferred_element_type=jnp.float32)
    m_x0, m_y0, m_x1, m_y1 = matched[0], matched[1], matched[2], matched[3]

    g_cx = ((m_x0 + m_x1) / 2.0 - pcx) / (_VAR0 * pw)
    g_cy = ((m_y0 + m_y1) / 2.0 - pcy) / (_VAR0 * ph)
    g_w = jnp.log((m_x1 - m_x0) / pw) / _VAR1
    g_h = jnp.log((m_y1 - m_y0) / ph) / _VAR1

    lR = loc_ref[0]

    def smooth_l1(d):
        ad = jnp.abs(d)
        return jnp.where(ad < 1.0, 0.5 * d * d, ad - 0.5)

    sl = (smooth_l1(lR[0] - g_cx) + smooth_l1(lR[1] - g_cy)
          + smooth_l1(lR[2] - g_w) + smooth_l1(lR[3] - g_h))
    loss_l = jnp.sum(sl * posf)

    cR = conf_ref[0]
    c0, c1 = cR[0], cR[1]
    cmx = jnp.maximum(c0, c1)
    lse = jnp.log(jnp.exp(c0 - cmx) + jnp.exp(c1 - cmx)) + cmx
    ce = lse - jnp.where(pos, c1, c0)

    mneg_ref[b] = jnp.where(pos, 0.0, ce)
    npos = jnp.sum(posf)
    part_ref[b] = jnp.stack(
        [jnp.reshape(loss_l, (1,)), jnp.reshape(jnp.sum(ce * posf), (1,)),
         jnp.reshape(npos, (1,))], axis=0)

    @pl.when(b == B - 1)
    def _mine():
        mneg = mneg_ref[...]
        parts = part_ref[...]
        nposv = parts[:, 2:3, :]
        num_neg = jnp.minimum(
            jnp.float32(_NEG_POS_RATIO) * nposv, jnp.float32(P - 1))

        mbits = lax.bitcast_convert_type(mneg, jnp.int32)
        res = jnp.zeros((B, 1, 1), jnp.int32)
        for bit in range(30, -1, -1):
            cand_t = res | jnp.int32(1 << bit)
            cnt = jnp.sum((mbits >= cand_t).astype(jnp.float32),
                          axis=2, keepdims=True)
            cnt = jnp.sum(cnt, axis=1, keepdims=True)
            res = jnp.where(cnt >= num_neg, cand_t, res)

        t_val = lax.bitcast_convert_type(res, jnp.float32)
        gt_mask = mbits > res
        sum_gt = jnp.sum(jnp.sum(jnp.where(gt_mask, mneg, 0.0),
                                 axis=2, keepdims=True), axis=1, keepdims=True)
        c_gt = jnp.sum(jnp.sum(gt_mask.astype(jnp.float32),
                               axis=2, keepdims=True), axis=1, keepdims=True)
        neg_contrib = sum_gt + (num_neg - c_gt) * t_val

        loss_c_rows = parts[:, 1:2, :] + neg_contrib
        n = jnp.sum(nposv)
        out_l_ref[...] = jnp.reshape(jnp.sum(parts[:, 0:1, :]) / n, (1, 1))
        out_c_ref[...] = jnp.reshape(jnp.sum(loss_c_rows) / n, (1, 1))


def kernel(loc_data, conf_data, priors, targets):
    B, P, _ = loc_data.shape
    NOBJ = targets.shape[1]
    SUB = _SUB
    LN = P // SUB

    loc_R = jnp.transpose(loc_data, (0, 2, 1)).reshape(B, 4, SUB, LN)
    conf_R = jnp.transpose(conf_data, (0, 2, 1)).reshape(B, 2, SUB, LN)
    priors_R = priors[:P].T.reshape(4, SUB, LN)
    truths = targets[:, :, :4]
    truths_T = jnp.transpose(truths, (0, 2, 1))

    f32 = jnp.float32
    loss_l, loss_c = pl.pallas_call(
        _row_kernel,
        grid=(B,),
        in_specs=[
            pl.BlockSpec((1, NOBJ, 4), lambda b: (b, 0, 0),
                         memory_space=pltpu.SMEM),
            pl.BlockSpec((1, 4, NOBJ), lambda b: (b, 0, 0)),
            pl.BlockSpec((4, SUB, LN), lambda b: (0, 0, 0)),
            pl.BlockSpec((1, 4, SUB, LN), lambda b: (b, 0, 0, 0)),
            pl.BlockSpec((1, 2, SUB, LN), lambda b: (b, 0, 0, 0)),
        ],
        out_specs=[
            pl.BlockSpec((1, 1), lambda b: (0, 0)),
            pl.BlockSpec((1, 1), lambda b: (0, 0)),
        ],
        out_shape=[
            jax.ShapeDtypeStruct((1, 1), f32),
            jax.ShapeDtypeStruct((1, 1), f32),
        ],
        scratch_shapes=[
            pltpu.VMEM((B, SUB, LN), f32),
            pltpu.VMEM((B, 3, 1), f32),
        ],
    )(truths, truths_T, priors_R, loc_R, conf_R)

    return loss_l[0, 0], loss_c[0, 0]


# speedup vs baseline: 2.1355x; 2.1355x over previous
import jax
import jax.numpy as jnp
from jax import lax
from jax.experimental import pallas as pl
from jax.experimental.pallas import tpu as pltpu

_OVERLAP_THRESH = 0.5
_NEG_POS_RATIO = 3
_VAR0, _VAR1 = 0.1, 0.2

_SUB = 8


def _row_kernel(truths_ref, truths_t_ref, priors_ref, loc_ref, conf_ref,
                out_l_ref, out_c_ref,
                mneg_ref, part_ref):
    b = pl.program_id(0)
    B = pl.num_programs(0)
    t = truths_ref[0]
    tT = truths_t_ref[0]
    pR = priors_ref[...]
    NOBJ = t.shape[0]
    SUB, LN = pR.shape[1], pR.shape[2]
    P = SUB * LN

    pcx, pcy, pw, ph = pR[0], pR[1], pR[2], pR[3]
    px0 = pcx - pw / 2.0
    py0 = pcy - ph / 2.0
    px1 = pcx + pw / 2.0
    py1 = pcy + ph / 2.0
    area_p = (px1 - px0) * (py1 - py0)

    t_x0 = jnp.reshape(t[:, 0:1], (NOBJ, 1, 1))
    t_y0 = jnp.reshape(t[:, 1:2], (NOBJ, 1, 1))
    t_x1 = jnp.reshape(t[:, 2:3], (NOBJ, 1, 1))
    t_y1 = jnp.reshape(t[:, 3:4], (NOBJ, 1, 1))
    area_t = (t_x1 - t_x0) * (t_y1 - t_y0)

    iw = jnp.maximum(jnp.minimum(t_x1, px1[None]) - jnp.maximum(t_x0, px0[None]), 0.0)
    ih = jnp.maximum(jnp.minimum(t_y1, py1[None]) - jnp.maximum(t_y0, py0[None]), 0.0)
    inter = iw * ih
    ov = inter / ((area_t + area_p[None]) - inter)

    o_iota = lax.broadcasted_iota(jnp.int32, (NOBJ, SUB, LN), 0)
    p_idx = (lax.broadcasted_iota(jnp.int32, (SUB, LN), 0) * LN
             + lax.broadcasted_iota(jnp.int32, (SUB, LN), 1))

    bto0 = jnp.max(ov, axis=0)
    bti0 = jnp.min(jnp.where(ov == bto0[None], o_iota, NOBJ), axis=0)

    maxv = jnp.max(jnp.max(ov, axis=2, keepdims=True), axis=1, keepdims=True)
    cand = jnp.where(ov == maxv, p_idx[None], P)
    bpi = jnp.min(jnp.min(cand, axis=2, keepdims=True), axis=1, keepdims=True)

    scat_o = jnp.max(jnp.where(p_idx[None] == bpi, o_iota, -1), axis=0)
    scat_any = scat_o >= 0
    bti = jnp.where(scat_any, scat_o, bti0)

    pos = scat_any | jnp.logical_not(bto0 < _OVERLAP_THRESH)
    posf = pos.astype(jnp.float32)

    onehot = (o_iota == bti[None]).astype(jnp.float32)
    matched = lax.dot_general(tT, onehot,
                              dimension_numbers=(((1,), (0,)), ((), ())),
                              preferred_element_type=jnp.float32)
    m_x0, m_y0, m_x1, m_y1 = matched[0], matched[1], matched[2], matched[3]

    g_cx = ((m_x0 + m_x1) / 2.0 - pcx) / (_VAR0 * pw)
    g_cy = ((m_y0 + m_y1) / 2.0 - pcy) / (_VAR0 * ph)
    g_w = jnp.log((m_x1 - m_x0) / pw) / _VAR1
    g_h = jnp.log((m_y1 - m_y0) / ph) / _VAR1

    lR = loc_ref[0]

    def smooth_l1(d):
        ad = jnp.abs(d)
        return jnp.where(ad < 1.0, 0.5 * d * d, ad - 0.5)

    sl = (smooth_l1(lR[0] - g_cx) + smooth_l1(lR[1] - g_cy)
          + smooth_l1(lR[2] - g_w) + smooth_l1(lR[3] - g_h))
    loss_l = jnp.sum(sl * posf)

    cR = conf_ref[0]
    c0, c1 = cR[0], cR[1]
    cmx = jnp.maximum(c0, c1)
    lse = jnp.log(jnp.exp(c0 - cmx) + jnp.exp(c1 - cmx)) + cmx
    ce = lse - jnp.where(pos, c1, c0)

    mneg_ref[b] = jnp.where(pos, 0.0, ce)
    npos = jnp.sum(posf)
    part_ref[b] = jnp.stack(
        [jnp.reshape(loss_l, (1,)), jnp.reshape(jnp.sum(ce * posf), (1,)),
         jnp.reshape(npos, (1,))], axis=0)

    @pl.when(b == B - 1)
    def _mine():
        mneg = mneg_ref[...]
        parts = part_ref[...]
        nposv = parts[:, 2:3, :]
        num_neg = jnp.minimum(
            jnp.float32(_NEG_POS_RATIO) * nposv, jnp.float32(P - 1))

        mbits = lax.bitcast_convert_type(mneg, jnp.int32)
        res = jnp.zeros((B, 1, 1), jnp.int32)
        for bit in range(30, -1, -1):
            cand_t = res | jnp.int32(1 << bit)
            cnt = jnp.sum((mbits >= cand_t).astype(jnp.float32),
                          axis=2, keepdims=True)
            cnt = jnp.sum(cnt, axis=1, keepdims=True)
            res = jnp.where(cnt >= num_neg, cand_t, res)

        t_val = lax.bitcast_convert_type(res, jnp.float32)
        gt_mask = mbits > res
        sum_gt = jnp.sum(jnp.sum(jnp.where(gt_mask, mneg, 0.0),
                                 axis=2, keepdims=True), axis=1, keepdims=True)
        c_gt = jnp.sum(jnp.sum(gt_mask.astype(jnp.float32),
                               axis=2, keepdims=True), axis=1, keepdims=True)
        neg_contrib = sum_gt + (num_neg - c_gt) * t_val

        loss_c_rows = parts[:, 1:2, :] + neg_contrib
        n = jnp.sum(nposv)
        out_l_ref[...] = jnp.reshape(jnp.sum(parts[:, 0:1, :]) / n, (1, 1))
        out_c_ref[...] = jnp.reshape(jnp.sum(loss_c_rows) / n, (1, 1))


def kernel(loc_data, conf_data, priors, targets):
    B, P, _ = loc_data.shape
    NOBJ = targets.shape[1]
    SUB = _SUB
    LN = P // SUB

    loc_R = jnp.transpose(loc_data, (0, 2, 1)).reshape(B, 4, SUB, LN)
    conf_R = jnp.transpose(conf_data, (0, 2, 1)).reshape(B, 2, SUB, LN)
    priors_R = priors[:P].T.reshape(4, SUB, LN)
    truths = targets[:, :, :4]
    truths_T = jnp.transpose(truths, (0, 2, 1))

    f32 = jnp.float32
    loss_l, loss_c = pl.pallas_call(
        _row_kernel,
        grid=(B,),
        in_specs=[
            pl.BlockSpec((1, NOBJ, 4), lambda b: (b, 0, 0)),
            pl.BlockSpec((1, 4, NOBJ), lambda b: (b, 0, 0)),
            pl.BlockSpec((4, SUB, LN), lambda b: (0, 0, 0)),
            pl.BlockSpec((1, 4, SUB, LN), lambda b: (b, 0, 0, 0)),
            pl.BlockSpec((1, 2, SUB, LN), lambda b: (b, 0, 0, 0)),
        ],
        out_specs=[
            pl.BlockSpec((1, 1), lambda b: (0, 0)),
            pl.BlockSpec((1, 1), lambda b: (0, 0)),
        ],
        out_shape=[
            jax.ShapeDtypeStruct((1, 1), f32),
            jax.ShapeDtypeStruct((1, 1), f32),
        ],
        scratch_shapes=[
            pltpu.VMEM((B, SUB, LN), f32),
            pltpu.VMEM((B, 3, 1), f32),
        ],
    )(truths, truths_T, priors_R, loc_R, conf_R)

    return loss_l[0, 0], loss_c[0, 0]
